# Initial kernel scaffold; baseline (speedup 1.0000x reference)
import jax
import jax.numpy as jnp
from jax.experimental import pallas as pl


def kernel(x, edge_index, edge_attr, st_types_feats, batch, st_table, Wl1, bl1, Wr1, Wl2, bl2, Wr2, Wlin, blin):
    raise NotImplementedError("write your pallas kernel here")


# speedup vs baseline: 5.0436x; 5.0436x over previous
import functools

import jax
import jax.numpy as jnp
from jax import lax
from jax.experimental import pallas as pl
from jax.experimental.pallas import tpu as pltpu
from jax.experimental.pallas import tpu_sc as plsc

_N = 10000
_E = 320000
_IN = 128
_EMB = 12
_H = 128
_OUT = 10
_NST = 256
_G = 64

_NWORK = 32
_EW = _E // _NWORK
_CH = 80
_NCH = _EW // _CH
_BN = 1000
_NB = _N // _BN

_SC_MESH = dict(core_axis_name="c", subcore_axis_name="s")


@functools.cache
def _make_sc_agg():
    @functools.partial(
        pl.kernel,
        mesh=plsc.VectorSubcoreMesh(**_SC_MESH),
        out_type=jax.ShapeDtypeStruct((2, _N, _H), jnp.float32),
        scratch_types=[
            pltpu.VMEM((_CH,), jnp.int32),
            pltpu.VMEM((_CH,), jnp.int32),
            pltpu.VMEM((_CH, _H), jnp.float32),
            pltpu.VMEM_SHARED((_N, _H), jnp.float32),
            pltpu.SemaphoreType.DMA,
        ],
    )
    def _sc_agg(src_hbm, dst_hbm, tbl_hbm, zeros_hbm, out_hbm,
                src_v, dst_v, rows_v, agg_sh, sem):
        c = lax.axis_index("c")
        s = lax.axis_index("s")
        wid = c * 16 + s

        @pl.when(s == 0)
        def _zero():
            pltpu.sync_copy(zeros_hbm, agg_sh)

        plsc.subcore_barrier()

        base = wid * _EW

        def body(i, carry):
            off = pl.multiple_of(base + i * _CH, 8)
            pltpu.sync_copy(src_hbm.at[pl.ds(off, _CH)], src_v)
            pltpu.sync_copy(dst_hbm.at[pl.ds(off, _CH)], dst_v)
            pltpu.async_copy(tbl_hbm.at[src_v], rows_v, sem).wait()
            pltpu.sync_copy(rows_v, agg_sh.at[dst_v], add=True)
            return carry

        lax.fori_loop(0, _NCH, body, 0)

        plsc.subcore_barrier()

        @pl.when(s == 0)
        def _writeout():
            pltpu.sync_copy(agg_sh, out_hbm.at[c])

    return _sc_agg


@functools.cache
def _make_sc_count():
    @functools.partial(
        pl.kernel,
        mesh=plsc.VectorSubcoreMesh(**_SC_MESH),
        out_type=jax.ShapeDtypeStruct((2, _N, _H), jnp.float32),
        scratch_types=[
            pltpu.VMEM((_CH,), jnp.int32),
            pltpu.VMEM((_CH, _H), jnp.float32),
            pltpu.VMEM_SHARED((_N, _H), jnp.float32),
        ],
    )
    def _sc_count(dst_hbm, ones_hbm, zeros_hbm, out_hbm,
                  dst_v, ones_v, cnt_sh):
        c = lax.axis_index("c")
        s = lax.axis_index("s")
        wid = c * 16 + s

        @pl.when(s == 0)
        def _zero():
            pltpu.sync_copy(zeros_hbm, cnt_sh)

        pltpu.sync_copy(ones_hbm, ones_v)
        plsc.subcore_barrier()

        base = wid * _EW

        def body(i, carry):
            off = pl.multiple_of(base + i * _CH, 8)
            pltpu.sync_copy(dst_hbm.at[pl.ds(off, _CH)], dst_v)
            pltpu.sync_copy(ones_v, cnt_sh.at[dst_v], add=True)
            return carry

        lax.fori_loop(0, _NCH, body, 0)

        plsc.subcore_barrier()

        @pl.when(s == 0)
        def _writeout():
            pltpu.sync_copy(cnt_sh, out_hbm.at[c])

    return _sc_count


def _tc1_body(x_ref, st_ref, tbl_ref, wlx_ref, wle_ref, wrx_ref, wre_ref,
              hl_ref, hr_ref):
    st = st_ref[...]
    oh = (st == lax.broadcasted_iota(jnp.int32, (1, _NST), 1)).astype(jnp.float32)
    emb = jnp.dot(oh, tbl_ref[...], preferred_element_type=jnp.float32)
    xb = x_ref[...]
    hl_ref[...] = (jnp.dot(xb, wlx_ref[...], preferred_element_type=jnp.float32)
                   + jnp.dot(emb, wle_ref[...], preferred_element_type=jnp.float32))
    hr_ref[...] = (jnp.dot(xb, wrx_ref[...], preferred_element_type=jnp.float32)
                   + jnp.dot(emb, wre_ref[...], preferred_element_type=jnp.float32))


def _combine(ea_ref, eb_ref, ca_ref, cb_ref, hr_ref, b_ref):
    sagg = ea_ref[...] + eb_ref[...]
    cnt = ca_ref[...][:, 0:1] + cb_ref[...][:, 0:1]
    recip = 1.0 / jnp.maximum(cnt, 1.0)
    return jnp.maximum(sagg * recip + b_ref[...] + hr_ref[...], 0.0)


def _tc2_body(ea_ref, eb_ref, ca_ref, cb_ref, hr_ref, b_ref, wl_ref, wr_ref,
              hlo_ref, hro_ref):
    h = _combine(ea_ref, eb_ref, ca_ref, cb_ref, hr_ref, b_ref)
    hlo_ref[...] = jnp.dot(h, wl_ref[...], preferred_element_type=jnp.float32)
    hro_ref[...] = jnp.dot(h, wr_ref[...], preferred_element_type=jnp.float32)


def _tc3_body(ea_ref, eb_ref, ca_ref, cb_ref, hr_ref, b_ref, bt_ref,
              wlin_ref, blin_ref, out_ref, pooled_acc, cnt_acc):
    i = pl.program_id(0)

    @pl.when(i == 0)
    def _init():
        pooled_acc[...] = jnp.zeros_like(pooled_acc)
        cnt_acc[...] = jnp.zeros_like(cnt_acc)

    h = _combine(ea_ref, eb_ref, ca_ref, cb_ref, hr_ref, b_ref)
    btT = bt_ref[0]
    ohT = (btT == lax.broadcasted_iota(jnp.int32, (_G, 1), 0)).astype(jnp.float32)
    pooled_acc[...] += jnp.dot(ohT, h, preferred_element_type=jnp.float32)
    cnt_acc[...] += jnp.broadcast_to(
        jnp.sum(ohT, axis=1, keepdims=True), (_G, _H))

    @pl.when(i == pl.num_programs(0) - 1)
    def _fin():
        pooled = pooled_acc[...] / jnp.maximum(cnt_acc[...], 1.0)
        out_ref[...] = (jnp.dot(pooled, wlin_ref[...],
                                preferred_element_type=jnp.float32)
                        + blin_ref[...])


def _row_spec(w):
    return pl.BlockSpec((_BN, w), lambda i: (i, 0))


def _full(shape):
    return pl.BlockSpec(shape, lambda i: tuple(0 for _ in shape))


_tc1 = pl.pallas_call(
    _tc1_body,
    grid=(_NB,),
    in_specs=[
        _row_spec(_IN),
        _row_spec(1),
        _full((_NST, _EMB)),
        _full((_IN, _H)),
        _full((_EMB, _H)),
        _full((_IN, _H)),
        _full((_EMB, _H)),
    ],
    out_specs=[_row_spec(_H), _row_spec(_H)],
    out_shape=[
        jax.ShapeDtypeStruct((_N, _H), jnp.float32),
        jax.ShapeDtypeStruct((_N, _H), jnp.float32),
    ],
)

_combine_specs = [
    _row_spec(_H),
    _row_spec(_H),
    _row_spec(_H),
    _row_spec(_H),
    _row_spec(_H),
    _full((1, _H)),
]

_tc2 = pl.pallas_call(
    _tc2_body,
    grid=(_NB,),
    in_specs=_combine_specs + [
        _full((_H, _H)),
        _full((_H, _H)),
    ],
    out_specs=[_row_spec(_H), _row_spec(_H)],
    out_shape=[
        jax.ShapeDtypeStruct((_N, _H), jnp.float32),
        jax.ShapeDtypeStruct((_N, _H), jnp.float32),
    ],
)

_tc3 = pl.pallas_call(
    _tc3_body,
    grid=(_NB,),
    in_specs=_combine_specs + [
        pl.BlockSpec((1, 1, _BN), lambda i: (i, 0, 0)),
        _full((_H, _OUT)),
        _full((1, _OUT)),
    ],
    out_specs=_full((_G, _OUT)),
    out_shape=jax.ShapeDtypeStruct((_G, _OUT), jnp.float32),
    scratch_shapes=[
        pltpu.VMEM((_G, _H), jnp.float32),
        pltpu.VMEM((_G, _H), jnp.float32),
    ],
)


def kernel(x, edge_index, edge_attr, st_types_feats, batch, st_table,
           Wl1, bl1, Wr1, Wl2, bl2, Wr2, Wlin, blin):
    src = edge_index[0]
    dst = edge_index[1]
    wl1t = Wl1.T
    wr1t = Wr1.T
    zeros = jnp.zeros((_N, _H), dtype=jnp.float32)
    ones = jnp.ones((_CH, _H), dtype=jnp.float32)
    batch3 = batch.reshape(_NB, 1, _BN)

    sc_agg = _make_sc_agg()
    sc_count = _make_sc_count()

    cnt = sc_count(dst, ones, zeros)
    hl0, hr0 = _tc1(x, st_types_feats, st_table,
                    wl1t[:_IN], wl1t[_IN:], wr1t[:_IN], wr1t[_IN:])
    agg0 = sc_agg(src, dst, hl0, zeros)
    hl1, hr1 = _tc2(agg0[0], agg0[1], cnt[0], cnt[1], hr0,
                    bl1.reshape(1, _H), Wl2.T, Wr2.T)
    agg1 = sc_agg(src, dst, hl1, zeros)
    logits = _tc3(agg1[0], agg1[1], cnt[0], cnt[1], hr1,
                  bl2.reshape(1, _H), batch3, Wlin.T, blin.reshape(1, _OUT))
    return logits
